# Initial kernel scaffold; baseline (speedup 1.0000x reference)
import jax
import jax.numpy as jnp
from jax.experimental import pallas as pl


def kernel(h, msg, r_label, msg_type, msg_r_label, self_loop_weight, relational_Q, relational_K, relational_V, ffn_w, ffn_b):
    raise NotImplementedError("write your pallas kernel here")


# speedup vs baseline: 6.6072x; 6.6072x over previous
import functools
import math

import jax
import jax.numpy as jnp
from jax import lax
from jax.experimental import pallas as pl

EMB = 32
ATT = 16
DEG = 16
NREL = 100
RPAD = 128

NODES = 40
SLOTS = NODES * (DEG + 1)


def _attn_block(h_ref, msg_ref, rl_ref, mrl_ref, mt_ref, wself_ref,
                q_ref, k_ref, v_ref, ffnwt_ref, ffnb_ref, out_ref):
    f32 = jnp.float32

    curr = jnp.dot(h_ref[...], wself_ref[...], preferred_element_type=f32)
    e_all = jnp.concatenate([curr, msg_ref[...]], axis=0)

    rl = rl_ref[0]
    mrl = mrl_ref[0]
    mt = mt_ref[0]
    idx_qv = jnp.concatenate([rl, mrl], axis=0)
    idx_k = jnp.concatenate([jnp.full((NODES, 1), NREL, rl.dtype), mt], axis=0)

    lane_r = lax.broadcasted_iota(jnp.int32, (SLOTS, RPAD), 1)
    o_qv = (idx_qv == lane_r).astype(f32)
    o_k = (idx_k == lane_r).astype(f32)

    r_sub = lax.broadcasted_iota(jnp.int32, (EMB, EMB * ATT), 0)
    r_lane = lax.broadcasted_iota(jnp.int32, (EMB, EMB * ATT), 1)
    rep_mat = ((r_lane // ATT) == r_sub).astype(f32)
    f_sub = lax.broadcasted_iota(jnp.int32, (EMB * ATT, ATT), 0)
    f_lane = lax.broadcasted_iota(jnp.int32, (EMB * ATT, ATT), 1)
    fold_mat = ((f_sub % ATT) == f_lane).astype(f32)

    e_rep = jnp.dot(e_all, rep_mat, preferred_element_type=f32)

    def project(onehot, tbl_ref):
        wg = jnp.dot(onehot, tbl_ref[...], preferred_element_type=f32)
        return jnp.dot(wg * e_rep, fold_mat, preferred_element_type=f32)

    q_all = project(o_qv, q_ref)
    k_all = project(o_k, k_ref)
    v_all = project(o_qv, v_ref)

    s_full = lax.dot_general(q_all, k_all, (((1,), (1,)), ((), ())),
                             preferred_element_type=f32) * (1.0 / math.sqrt(ATT))

    i_sub = lax.broadcasted_iota(jnp.int32, (SLOTS, 1), 0)
    i_lane = lax.broadcasted_iota(jnp.int32, (1, SLOTS), 1)
    node_s = jnp.where(i_sub < NODES, i_sub, (i_sub - NODES) >> 4)
    node_t = jnp.where(i_lane < NODES, i_lane, (i_lane - NODES) >> 4)
    mask = node_s == node_t

    neg = jnp.where(mask, s_full, -1e30)
    m = jnp.max(neg, axis=0, keepdims=True)
    ex = jnp.exp(neg - m)
    denom = jnp.sum(ex, axis=0, keepdims=True)
    is_self = (i_sub < NODES).astype(f32)
    numer = jnp.sum(ex * is_self, axis=0, keepdims=True)
    attn0 = numer / denom

    n_sub = lax.broadcasted_iota(jnp.int32, (NODES, 1), 0)
    sel = (node_t == n_sub).astype(f32) * attn0
    pooled = jnp.dot(sel, v_all, preferred_element_type=f32)

    out_ref[...] = (jnp.dot(pooled, ffnwt_ref[...], preferred_element_type=f32)
                    + ffnb_ref[...])


@functools.partial(jax.jit, static_argnames=())
def kernel(h, msg, r_label, msg_type, msg_r_label, self_loop_weight,
           relational_Q, relational_K, relational_V, ffn_w, ffn_b):
    bnum = h.shape[0]
    nblk = bnum // NODES
    inp = h.shape[1]

    msg2d = msg.reshape(bnum * DEG, EMB)
    rl3 = r_label.astype(jnp.int32).reshape(nblk, NODES, 1)
    mrl3 = msg_r_label.astype(jnp.int32).reshape(nblk, NODES * DEG, 1)
    mt3 = msg_type.astype(jnp.int32).reshape(nblk, NODES * DEG, 1)

    def padtbl(t):
        flat = t.reshape(NREL + 1, EMB * ATT)
        return jnp.concatenate(
            [flat, jnp.zeros((RPAD - (NREL + 1), EMB * ATT), flat.dtype)], axis=0)

    qt, kt, vt = padtbl(relational_Q), padtbl(relational_K), padtbl(relational_V)
    ffn_wt = ffn_w.T
    ffn_b2 = ffn_b.reshape(1, EMB)

    full = lambda shape: pl.BlockSpec(shape, lambda i: (0,) * len(shape))
    out = pl.pallas_call(
        _attn_block,
        grid=(nblk,),
        in_specs=[
            pl.BlockSpec((NODES, inp), lambda i: (i, 0)),
            pl.BlockSpec((NODES * DEG, EMB), lambda i: (i, 0)),
            pl.BlockSpec((1, NODES, 1), lambda i: (i, 0, 0)),
            pl.BlockSpec((1, NODES * DEG, 1), lambda i: (i, 0, 0)),
            pl.BlockSpec((1, NODES * DEG, 1), lambda i: (i, 0, 0)),
            full((inp, EMB)),
            full((RPAD, EMB * ATT)),
            full((RPAD, EMB * ATT)),
            full((RPAD, EMB * ATT)),
            full((ATT, EMB)),
            full((1, EMB)),
        ],
        out_specs=pl.BlockSpec((NODES, EMB), lambda i: (i, 0)),
        out_shape=jax.ShapeDtypeStruct((bnum, EMB), jnp.float32),
    )(h, msg2d, rl3, mrl3, mt3, self_loop_weight, qt, kt, vt, ffn_wt, ffn_b2)
    return out
